# Initial kernel scaffold; baseline (speedup 1.0000x reference)
import jax
import jax.numpy as jnp
from jax.experimental import pallas as pl


def kernel(points, W1, g1, b1, W2, g2, b2, W3, g3, b3, W4, g4, b4, W5, g5, b5, Ws1, gs1, bs1, Ws2, gs2, bs2, Ws3, bias3):
    raise NotImplementedError("write your pallas kernel here")


# speedup vs baseline: 7.7259x; 7.7259x over previous
import functools

import jax
import jax.numpy as jnp
from jax.experimental import pallas as pl
from jax.experimental.pallas import tpu as pltpu

KNN = 20
EPS = 1e-5
NEG_INF = float('-inf')


def _tree(op, xs):
    xs = list(xs)
    while len(xs) > 1:
        nxt = []
        for i in range(0, len(xs) - 1, 2):
            nxt.append(op(xs[i], xs[i + 1]))
        if len(xs) % 2:
            nxt.append(xs[-1])
        xs = nxt
    return xs[0]


def _lrelu(x):
    return jnp.where(x >= 0, x, 0.2 * x)


def _bf16r(x):
    return x.astype(jnp.bfloat16).astype(jnp.float32)


def _edge_kernel(N, C, O, MT, T, K8,
                 xt_ref, xc_ref, wt_ref,
                 zmax_ref, zmin_ref, ssum_ref, ssq_ref,
                 nd_scr, x3_scr, xxr_scr, idxv_scr, idx_smem,
                 f3_scr, y_scr, sem):
    t = pl.program_id(1)

    @pl.when(t == 0)
    def _():
        xt = xt_ref[0]
        x3_scr[...] = xt.reshape(N, 1, C)
        acc = xc_ref[0, 0:1, :] * xc_ref[0, 0:1, :]
        for c in range(1, C):
            row = xc_ref[0, c:c + 1, :]
            acc = acc + row * row
        xxr_scr[...] = acc
        ssum_ref[...] = jnp.zeros_like(ssum_ref)
        ssq_ref[...] = jnp.zeros_like(ssq_ref)

    base = pl.multiple_of(t * MT, MT)
    xt_t = xt_ref[0, pl.ds(base, MT), :]

    inner = jax.lax.dot_general(xt_t, xt_ref[0], (((1,), (1,)), ((), ())),
                                preferred_element_type=jnp.float32)
    xx_i = jnp.sum(xt_t * xt_t, axis=1, keepdims=True)
    nd_scr[...] = (2.0 * inner - xx_i) - xxr_scr[...]

    iota = jax.lax.broadcasted_iota(jnp.int32, (MT, N), 1)
    for r in range(KNN):
        nd = nd_scr[...]
        idx = jnp.argmax(nd, axis=-1).astype(jnp.int32)
        idxv_scr[r, :] = idx
        if r != KNN - 1:
            nd_scr[...] = jnp.where(iota == idx[:, None], NEG_INF, nd)

    cp = pltpu.make_async_copy(idxv_scr, idx_smem, sem)
    cp.start()
    cp.wait()

    def body(m, _):
        xm = x3_scr[base + m, 0]
        xmq = _bf16r(xm)
        for r in range(KNN):
            xj = x3_scr[idx_smem[r, m], 0]
            fr = jnp.concatenate([_bf16r(xj - xm), xmq])
            f3_scr[r * MT + m] = fr.reshape(1, 2 * C)
        return 0

    jax.lax.fori_loop(0, MT, body, 0)

    y_scr[...] = jnp.dot(f3_scr[...].reshape(KNN * MT, 2 * C), wt_ref[...],
                         preferred_element_type=jnp.float32)

    MC = 32
    s1_parts = []
    s2_parts = []
    for mc in range(MT // MC):
        ys = [y_scr[r * MT + mc * MC:r * MT + mc * MC + MC, :]
              for r in range(KNN)]
        zmax_ref[0, mc * MC:(mc + 1) * MC, :] = _tree(jnp.maximum, ys)
        zmin_ref[0, mc * MC:(mc + 1) * MC, :] = _tree(jnp.minimum, ys)
        s1c = _tree(jnp.add, ys)
        s2c = _tree(jnp.add, [y * y for y in ys])
        s1_parts.append(jnp.sum(s1c, axis=0, keepdims=True))
        s2_parts.append(jnp.sum(s2c, axis=0, keepdims=True))
    ssum_ref[...] += _tree(jnp.add, s1_parts)[None]
    ssq_ref[...] += _tree(jnp.add, s2_parts)[None]


def _edge_block(xt, wt, MT=128):
    B, N, C = xt.shape
    O = wt.shape[1]
    T = N // MT
    K8 = 24
    kernel = functools.partial(_edge_kernel, N, C, O, MT, T, K8)
    zmax, zmin, ssum, ssq = pl.pallas_call(
        kernel,
        grid=(B, T),
        in_specs=[
            pl.BlockSpec((1, N, C), lambda b, t: (b, 0, 0)),
            pl.BlockSpec((1, C, N), lambda b, t: (b, 0, 0)),
            pl.BlockSpec((2 * C, O), lambda b, t: (0, 0)),
        ],
        out_specs=[
            pl.BlockSpec((1, MT, O), lambda b, t: (b, t, 0)),
            pl.BlockSpec((1, MT, O), lambda b, t: (b, t, 0)),
            pl.BlockSpec((1, 1, O), lambda b, t: (b, 0, 0)),
            pl.BlockSpec((1, 1, O), lambda b, t: (b, 0, 0)),
        ],
        out_shape=[
            jax.ShapeDtypeStruct((B, N, O), jnp.float32),
            jax.ShapeDtypeStruct((B, N, O), jnp.float32),
            jax.ShapeDtypeStruct((B, 1, O), jnp.float32),
            jax.ShapeDtypeStruct((B, 1, O), jnp.float32),
        ],
        scratch_shapes=[
            pltpu.VMEM((MT, N), jnp.float32),
            pltpu.VMEM((N, 1, C), jnp.float32),
            pltpu.VMEM((1, N), jnp.float32),
            pltpu.VMEM((K8, MT), jnp.int32),
            pltpu.SMEM((K8, MT), jnp.int32),
            pltpu.VMEM((KNN * MT, 1, 2 * C), jnp.float32),
            pltpu.VMEM((KNN * MT, O), jnp.float32),
            pltpu.SemaphoreType.DMA,
        ],
        compiler_params=pltpu.CompilerParams(
            dimension_semantics=("parallel", "arbitrary"),
            vmem_limit_bytes=100 * 1024 * 1024,
        ),
        name=f"edge_knn_c{C}_o{O}",
    )(xt, jnp.transpose(xt, (0, 2, 1)), wt)
    return zmax, zmin, ssum, ssq


def _finish_kernel(cnt, zmax_ref, zmin_ref, st_ref, sq_ref, g_ref, b_ref,
                   out_ref):
    rc = 1.0 / cnt
    mean = st_ref[...] * rc
    var = sq_ref[...] * rc - mean * mean
    r = jax.lax.rsqrt(var + EPS)
    sgn = g_ref[...] * r
    z = jnp.where(sgn >= 0, zmax_ref[...], zmin_ref[...])
    out_ref[...] = _lrelu((z - mean) * r * g_ref[...] + b_ref[...])


def _edge_finish(zmax, zmin, ssum, ssq, g, b, cnt, M2=512):
    B, N, O = zmax.shape
    T = N // M2
    st = jnp.sum(ssum, axis=0).reshape(1, 1, O)
    sq = jnp.sum(ssq, axis=0).reshape(1, 1, O)
    g3 = g.reshape(1, 1, O).astype(jnp.float32)
    b3 = b.reshape(1, 1, O).astype(jnp.float32)
    kernel = functools.partial(_finish_kernel, float(cnt))
    return pl.pallas_call(
        kernel,
        grid=(B, T),
        in_specs=[
            pl.BlockSpec((1, M2, O), lambda b_, t: (b_, t, 0)),
            pl.BlockSpec((1, M2, O), lambda b_, t: (b_, t, 0)),
            pl.BlockSpec((1, 1, O), lambda b_, t: (0, 0, 0)),
            pl.BlockSpec((1, 1, O), lambda b_, t: (0, 0, 0)),
            pl.BlockSpec((1, 1, O), lambda b_, t: (0, 0, 0)),
            pl.BlockSpec((1, 1, O), lambda b_, t: (0, 0, 0)),
        ],
        out_specs=pl.BlockSpec((1, M2, O), lambda b_, t: (b_, t, 0)),
        out_shape=jax.ShapeDtypeStruct((B, N, O), jnp.float32),
        compiler_params=pltpu.CompilerParams(
            dimension_semantics=("parallel", "arbitrary"),
            vmem_limit_bytes=100 * 1024 * 1024,
        ),
        name=f"edge_finish_o{O}",
    )(zmax, zmin, st, sq, g3, b3)


def _h1_kernel(xc_ref, w_ref, ssum_ref, ssq_ref, ymax_ref, ymin_ref):
    t = pl.program_id(1)
    y = jnp.dot(xc_ref[0], w_ref[...], preferred_element_type=jnp.float32)

    @pl.when(t == 0)
    def _():
        ssum_ref[...] = jnp.zeros_like(ssum_ref)
        ssq_ref[...] = jnp.zeros_like(ssq_ref)
        ymax_ref[...] = jnp.full_like(ymax_ref, -jnp.inf)
        ymin_ref[...] = jnp.full_like(ymin_ref, jnp.inf)

    ssum_ref[...] += jnp.sum(y, axis=0, keepdims=True)[None]
    ssq_ref[...] += jnp.sum(y * y, axis=0, keepdims=True)[None]
    ymax_ref[...] = jnp.maximum(ymax_ref[...],
                                jnp.max(y, axis=0, keepdims=True)[None])
    ymin_ref[...] = jnp.minimum(ymin_ref[...],
                                jnp.min(y, axis=0, keepdims=True)[None])


def _head1(xc, w5t, MH=512):
    B, N, C = xc.shape
    O = w5t.shape[1]
    T = N // MH
    return pl.pallas_call(
        _h1_kernel,
        grid=(B, T),
        in_specs=[
            pl.BlockSpec((1, MH, C), lambda b, t: (b, t, 0)),
            pl.BlockSpec((C, O), lambda b, t: (0, 0)),
        ],
        out_specs=[
            pl.BlockSpec((1, 1, O), lambda b, t: (b, 0, 0)),
            pl.BlockSpec((1, 1, O), lambda b, t: (b, 0, 0)),
            pl.BlockSpec((1, 1, O), lambda b, t: (b, 0, 0)),
            pl.BlockSpec((1, 1, O), lambda b, t: (b, 0, 0)),
        ],
        out_shape=[jax.ShapeDtypeStruct((B, 1, O), jnp.float32)] * 4,
        compiler_params=pltpu.CompilerParams(
            dimension_semantics=("parallel", "arbitrary"),
            vmem_limit_bytes=100 * 1024 * 1024,
        ),
        name="head1_stats",
    )(xc, w5t)


def _h2_kernel(cnt, xc_ref, at_ref, gt_ref, ymax_ref, ymin_ref,
               st_ref, sq_ref, g5_ref, b5_ref,
               hpre_ref, ssum_ref, ssq_ref):
    t = pl.program_id(1)
    rc = 1.0 / cnt
    mean = st_ref[...] * rc
    var = sq_ref[...] * rc - mean * mean
    r = jax.lax.rsqrt(var + EPS)
    sgn = g5_ref[...] * r
    ysel = jnp.where(sgn >= 0, ymax_ref[...], ymin_ref[...])
    gvec = _lrelu((ysel - mean) * r * g5_ref[...] + b5_ref[...])[0]
    c = jnp.dot(gvec, gt_ref[...],
                preferred_element_type=jnp.float32)
    hpre = jnp.dot(xc_ref[0], at_ref[...],
                   preferred_element_type=jnp.float32) + c

    @pl.when(t == 0)
    def _():
        ssum_ref[...] = jnp.zeros_like(ssum_ref)
        ssq_ref[...] = jnp.zeros_like(ssq_ref)

    ssum_ref[...] += jnp.sum(hpre, axis=0, keepdims=True)[None]
    ssq_ref[...] += jnp.sum(hpre * hpre, axis=0, keepdims=True)[None]
    hpre_ref[0] = hpre


def _head2(xc, at, gt, ymax, ymin, st, sq, g5, b5, cnt, MH=512):
    B, N, C = xc.shape
    O = at.shape[1]
    T = N // MH
    kernel = functools.partial(_h2_kernel, float(cnt))
    return pl.pallas_call(
        kernel,
        grid=(B, T),
        in_specs=[
            pl.BlockSpec((1, MH, C), lambda b, t: (b, t, 0)),
            pl.BlockSpec((C, O), lambda b, t: (0, 0)),
            pl.BlockSpec((C, O), lambda b, t: (0, 0)),
            pl.BlockSpec((1, 1, C), lambda b, t: (b, 0, 0)),
            pl.BlockSpec((1, 1, C), lambda b, t: (b, 0, 0)),
            pl.BlockSpec((1, 1, C), lambda b, t: (0, 0, 0)),
            pl.BlockSpec((1, 1, C), lambda b, t: (0, 0, 0)),
            pl.BlockSpec((1, 1, C), lambda b, t: (0, 0, 0)),
            pl.BlockSpec((1, 1, C), lambda b, t: (0, 0, 0)),
        ],
        out_specs=[
            pl.BlockSpec((1, MH, O), lambda b, t: (b, t, 0)),
            pl.BlockSpec((1, 1, O), lambda b, t: (b, 0, 0)),
            pl.BlockSpec((1, 1, O), lambda b, t: (b, 0, 0)),
        ],
        out_shape=[
            jax.ShapeDtypeStruct((B, N, O), jnp.float32),
            jax.ShapeDtypeStruct((B, 1, O), jnp.float32),
            jax.ShapeDtypeStruct((B, 1, O), jnp.float32),
        ],
        compiler_params=pltpu.CompilerParams(
            dimension_semantics=("parallel", "arbitrary"),
            vmem_limit_bytes=100 * 1024 * 1024,
        ),
        name="head2_mlp",
    )(xc, at, gt, ymax, ymin, st, sq, g5, b5)


def _h3_kernel(cnt, x_ref, w_ref, st_ref, sq_ref, g_ref, b_ref,
               y_ref, ssum_ref, ssq_ref):
    t = pl.program_id(1)
    rc = 1.0 / cnt
    mean = st_ref[...] * rc
    var = sq_ref[...] * rc - mean * mean
    r = jax.lax.rsqrt(var + EPS)
    h = _lrelu((x_ref[0] - mean[0]) * r[0] * g_ref[0] + b_ref[0])
    y = jnp.dot(h, w_ref[...], preferred_element_type=jnp.float32)

    @pl.when(t == 0)
    def _():
        ssum_ref[...] = jnp.zeros_like(ssum_ref)
        ssq_ref[...] = jnp.zeros_like(ssq_ref)

    ssum_ref[...] += jnp.sum(y, axis=0, keepdims=True)[None]
    ssq_ref[...] += jnp.sum(y * y, axis=0, keepdims=True)[None]
    y_ref[0] = y


def _head3(x, wt, st, sq, g, b, cnt, MH=512):
    B, N, C = x.shape
    O = wt.shape[1]
    T = N // MH
    kernel = functools.partial(_h3_kernel, float(cnt))
    return pl.pallas_call(
        kernel,
        grid=(B, T),
        in_specs=[
            pl.BlockSpec((1, MH, C), lambda b_, t: (b_, t, 0)),
            pl.BlockSpec((C, O), lambda b_, t: (0, 0)),
            pl.BlockSpec((1, 1, C), lambda b_, t: (0, 0, 0)),
            pl.BlockSpec((1, 1, C), lambda b_, t: (0, 0, 0)),
            pl.BlockSpec((1, 1, C), lambda b_, t: (0, 0, 0)),
            pl.BlockSpec((1, 1, C), lambda b_, t: (0, 0, 0)),
        ],
        out_specs=[
            pl.BlockSpec((1, MH, O), lambda b_, t: (b_, t, 0)),
            pl.BlockSpec((1, 1, O), lambda b_, t: (b_, 0, 0)),
            pl.BlockSpec((1, 1, O), lambda b_, t: (b_, 0, 0)),
        ],
        out_shape=[
            jax.ShapeDtypeStruct((B, N, O), jnp.float32),
            jax.ShapeDtypeStruct((B, 1, O), jnp.float32),
            jax.ShapeDtypeStruct((B, 1, O), jnp.float32),
        ],
        compiler_params=pltpu.CompilerParams(
            dimension_semantics=("parallel", "arbitrary"),
            vmem_limit_bytes=100 * 1024 * 1024,
        ),
        name=f"head3_mlp_c{C}",
    )(x, wt, st, sq, g, b)


def _h4_kernel(cnt, x_ref, w_ref, st_ref, sq_ref, g_ref, b_ref, bias_ref,
               y_ref):
    rc = 1.0 / cnt
    mean = st_ref[...] * rc
    var = sq_ref[...] * rc - mean * mean
    r = jax.lax.rsqrt(var + EPS)
    h = _lrelu((x_ref[0] - mean[0]) * r[0] * g_ref[0] + b_ref[0])
    y_ref[0] = jnp.dot(h, w_ref[...],
                       preferred_element_type=jnp.float32) + bias_ref[0]


def _head4(x, wt, st, sq, g, b, bias, cnt, MH=512):
    B, N, C = x.shape
    O = wt.shape[1]
    T = N // MH
    kernel = functools.partial(_h4_kernel, float(cnt))
    return pl.pallas_call(
        kernel,
        grid=(B, T),
        in_specs=[
            pl.BlockSpec((1, MH, C), lambda b_, t: (b_, t, 0)),
            pl.BlockSpec((C, O), lambda b_, t: (0, 0)),
            pl.BlockSpec((1, 1, C), lambda b_, t: (0, 0, 0)),
            pl.BlockSpec((1, 1, C), lambda b_, t: (0, 0, 0)),
            pl.BlockSpec((1, 1, C), lambda b_, t: (0, 0, 0)),
            pl.BlockSpec((1, 1, C), lambda b_, t: (0, 0, 0)),
            pl.BlockSpec((1, 1, O), lambda b_, t: (0, 0, 0)),
        ],
        out_specs=pl.BlockSpec((1, MH, O), lambda b_, t: (b_, t, 0)),
        out_shape=jax.ShapeDtypeStruct((B, N, O), jnp.float32),
        compiler_params=pltpu.CompilerParams(
            dimension_semantics=("parallel", "arbitrary"),
            vmem_limit_bytes=100 * 1024 * 1024,
        ),
        name="head4_out",
    )(x, wt, st, sq, g, b, bias)


def kernel(points, W1, g1, b1, W2, g2, b2, W3, g3, b3, W4, g4, b4,
           W5, g5, b5, Ws1, gs1, bs1, Ws2, gs2, bs2, Ws3, bias3):
    B, N, _ = points.shape
    cnt_e = float(B * N * KNN)
    cnt_n = float(B * N)

    xt = points.astype(jnp.float32)

    outs = []
    x_cur = xt
    for (W, g, b) in ((W1, g1, b1), (W2, g2, b2), (W3, g3, b3), (W4, g4, b4)):
        zmax, zmin, ssum, ssq = _edge_block(x_cur, W.T.astype(jnp.float32))
        x_cur = _edge_finish(zmax, zmin, ssum, ssq, g, b, cnt_e)
        outs.append(x_cur)

    xc = jnp.concatenate(outs, axis=2)

    s5sum, s5sq, ymax, ymin = _head1(xc, W5.T.astype(jnp.float32))
    st5 = jnp.sum(s5sum, axis=0).reshape(1, 1, 512)
    sq5 = jnp.sum(s5sq, axis=0).reshape(1, 1, 512)

    at = Ws1[:, :512].T.astype(jnp.float32)
    gt = Ws1[:, 512:].T.astype(jnp.float32)
    hpre, s1sum, s1sq = _head2(
        xc, at, gt, ymax, ymin, st5, sq5,
        g5.reshape(1, 1, 512).astype(jnp.float32),
        b5.reshape(1, 1, 512).astype(jnp.float32), cnt_n)
    st1 = jnp.sum(s1sum, axis=0).reshape(1, 1, 256)
    sq1 = jnp.sum(s1sq, axis=0).reshape(1, 1, 256)

    y2, s2sum, s2sq = _head3(
        hpre, Ws2.T.astype(jnp.float32), st1, sq1,
        gs1.reshape(1, 1, 256).astype(jnp.float32),
        bs1.reshape(1, 1, 256).astype(jnp.float32), cnt_n)
    st2 = jnp.sum(s2sum, axis=0).reshape(1, 1, 128)
    sq2 = jnp.sum(s2sq, axis=0).reshape(1, 1, 128)

    w3pad = jnp.zeros((128, 128), jnp.float32).at[:, :2].set(
        Ws3.T.astype(jnp.float32))
    bias_pad = jnp.zeros((1, 1, 128), jnp.float32).at[0, 0, :2].set(
        bias3.astype(jnp.float32))
    o = _head4(y2, w3pad, st2, sq2,
               gs2.reshape(1, 1, 128).astype(jnp.float32),
               bs2.reshape(1, 1, 128).astype(jnp.float32),
               bias_pad, cnt_n)

    return jnp.transpose(o[:, :, :2], (0, 2, 1))


# speedup vs baseline: 7.7304x; 1.0006x over previous
import functools

import jax
import jax.numpy as jnp
from jax.experimental import pallas as pl
from jax.experimental.pallas import tpu as pltpu

KNN = 20
EPS = 1e-5
NEG_INF = float('-inf')


def _tree(op, xs):
    xs = list(xs)
    while len(xs) > 1:
        nxt = []
        for i in range(0, len(xs) - 1, 2):
            nxt.append(op(xs[i], xs[i + 1]))
        if len(xs) % 2:
            nxt.append(xs[-1])
        xs = nxt
    return xs[0]


def _lrelu(x):
    return jnp.where(x >= 0, x, 0.2 * x)


def _bf16r(x):
    return x.astype(jnp.bfloat16).astype(jnp.float32)


def _edge_kernel(N, C, O, MT, T, K8,
                 xt_ref, xc_ref, wt_ref,
                 zmax_ref, zmin_ref, ssum_ref, ssq_ref,
                 nd_scr, x3_scr, xxr_scr, idxv_scr, idx_smem,
                 f3_scr, y_scr, sem):
    t = pl.program_id(1)

    @pl.when(t == 0)
    def _():
        xt = xt_ref[0]
        x3_scr[...] = xt.reshape(N, 1, C)
        acc = xc_ref[0, 0:1, :] * xc_ref[0, 0:1, :]
        for c in range(1, C):
            row = xc_ref[0, c:c + 1, :]
            acc = acc + row * row
        xxr_scr[...] = acc
        ssum_ref[...] = jnp.zeros_like(ssum_ref)
        ssq_ref[...] = jnp.zeros_like(ssq_ref)

    base = pl.multiple_of(t * MT, MT)
    xt_t = xt_ref[0, pl.ds(base, MT), :]

    inner = jax.lax.dot_general(xt_t, xt_ref[0], (((1,), (1,)), ((), ())),
                                preferred_element_type=jnp.float32)
    xx_i = jnp.sum(xt_t * xt_t, axis=1, keepdims=True)
    nd_scr[...] = (2.0 * inner - xx_i) - xxr_scr[...]

    iota = jax.lax.broadcasted_iota(jnp.int32, (MT, N), 1)
    for r in range(KNN):
        nd = nd_scr[...]
        idx = jnp.argmax(nd, axis=-1).astype(jnp.int32)
        idxv_scr[r, :] = idx
        if r != KNN - 1:
            nd_scr[...] = jnp.where(iota == idx[:, None], NEG_INF, nd)

    cp = pltpu.make_async_copy(idxv_scr, idx_smem, sem)
    cp.start()
    cp.wait()

    def body(m, _):
        xm = x3_scr[base + m, 0]
        xmq = _bf16r(xm)
        for r in range(KNN):
            xj = x3_scr[idx_smem[r, m], 0]
            fr = jnp.concatenate([_bf16r(xj - xm), xmq])
            f3_scr[r * MT + m] = fr.reshape(1, 2 * C)
        return 0

    jax.lax.fori_loop(0, MT, body, 0)

    y_scr[...] = jnp.dot(f3_scr[...].reshape(KNN * MT, 2 * C), wt_ref[...],
                         preferred_element_type=jnp.float32)

    MC = 32
    s1_parts = []
    s2_parts = []
    for mc in range(MT // MC):
        ys = [y_scr[r * MT + mc * MC:r * MT + mc * MC + MC, :]
              for r in range(KNN)]
        zmax_ref[0, mc * MC:(mc + 1) * MC, :] = _tree(jnp.maximum, ys)
        zmin_ref[0, mc * MC:(mc + 1) * MC, :] = _tree(jnp.minimum, ys)
        s1c = _tree(jnp.add, ys)
        s2c = _tree(jnp.add, [y * y for y in ys])
        s1_parts.append(jnp.sum(s1c, axis=0, keepdims=True))
        s2_parts.append(jnp.sum(s2c, axis=0, keepdims=True))
    ssum_ref[...] += _tree(jnp.add, s1_parts)[None]
    ssq_ref[...] += _tree(jnp.add, s2_parts)[None]


def _edge_block(xt, wt, MT=128):
    B, N, C = xt.shape
    O = wt.shape[1]
    T = N // MT
    K8 = 24
    kernel = functools.partial(_edge_kernel, N, C, O, MT, T, K8)
    zmax, zmin, ssum, ssq = pl.pallas_call(
        kernel,
        grid=(B, T),
        in_specs=[
            pl.BlockSpec((1, N, C), lambda b, t: (b, 0, 0)),
            pl.BlockSpec((1, C, N), lambda b, t: (b, 0, 0)),
            pl.BlockSpec((2 * C, O), lambda b, t: (0, 0)),
        ],
        out_specs=[
            pl.BlockSpec((1, MT, O), lambda b, t: (b, t, 0)),
            pl.BlockSpec((1, MT, O), lambda b, t: (b, t, 0)),
            pl.BlockSpec((1, 1, O), lambda b, t: (b, 0, 0)),
            pl.BlockSpec((1, 1, O), lambda b, t: (b, 0, 0)),
        ],
        out_shape=[
            jax.ShapeDtypeStruct((B, N, O), jnp.float32),
            jax.ShapeDtypeStruct((B, N, O), jnp.float32),
            jax.ShapeDtypeStruct((B, 1, O), jnp.float32),
            jax.ShapeDtypeStruct((B, 1, O), jnp.float32),
        ],
        scratch_shapes=[
            pltpu.VMEM((MT, N), jnp.float32),
            pltpu.VMEM((N, 1, C), jnp.float32),
            pltpu.VMEM((1, N), jnp.float32),
            pltpu.VMEM((K8, MT), jnp.int32),
            pltpu.SMEM((K8, MT), jnp.int32),
            pltpu.VMEM((KNN * MT, 1, 2 * C), jnp.float32),
            pltpu.VMEM((KNN * MT, O), jnp.float32),
            pltpu.SemaphoreType.DMA,
        ],
        compiler_params=pltpu.CompilerParams(
            dimension_semantics=("parallel", "arbitrary"),
            vmem_limit_bytes=100 * 1024 * 1024,
        ),
        name=f"edge_knn_c{C}_o{O}",
    )(xt, jnp.transpose(xt, (0, 2, 1)), wt)
    return zmax, zmin, ssum, ssq


def _finish_kernel(zmax_ref, zmin_ref, mean_ref, r_ref, g_ref, b_ref,
                   out_ref):
    mean = mean_ref[...]
    r = r_ref[...]
    sgn = g_ref[...] * r
    z = jnp.where(sgn >= 0, zmax_ref[...], zmin_ref[...])
    out_ref[...] = _lrelu((z - mean) * r * g_ref[...] + b_ref[...])


def _edge_finish(zmax, zmin, ssum, ssq, g, b, cnt, M2=512):
    B, N, O = zmax.shape
    T = N // M2
    mean = (jnp.sum(ssum, axis=0) / cnt).reshape(1, 1, O)
    var = (jnp.sum(ssq, axis=0) / cnt).reshape(1, 1, O) - mean * mean
    r = jax.lax.rsqrt(var + EPS)
    g3 = g.reshape(1, 1, O).astype(jnp.float32)
    b3 = b.reshape(1, 1, O).astype(jnp.float32)
    kernel = _finish_kernel
    return pl.pallas_call(
        kernel,
        grid=(B, T),
        in_specs=[
            pl.BlockSpec((1, M2, O), lambda b_, t: (b_, t, 0)),
            pl.BlockSpec((1, M2, O), lambda b_, t: (b_, t, 0)),
            pl.BlockSpec((1, 1, O), lambda b_, t: (0, 0, 0)),
            pl.BlockSpec((1, 1, O), lambda b_, t: (0, 0, 0)),
            pl.BlockSpec((1, 1, O), lambda b_, t: (0, 0, 0)),
            pl.BlockSpec((1, 1, O), lambda b_, t: (0, 0, 0)),
        ],
        out_specs=pl.BlockSpec((1, M2, O), lambda b_, t: (b_, t, 0)),
        out_shape=jax.ShapeDtypeStruct((B, N, O), jnp.float32),
        compiler_params=pltpu.CompilerParams(
            dimension_semantics=("parallel", "arbitrary"),
            vmem_limit_bytes=100 * 1024 * 1024,
        ),
        name=f"edge_finish_o{O}",
    )(zmax, zmin, mean, r, g3, b3)


def _h1_kernel(xc_ref, w_ref, ssum_ref, ssq_ref, ymax_ref, ymin_ref):
    t = pl.program_id(1)
    y = jnp.dot(xc_ref[0], w_ref[...], preferred_element_type=jnp.float32)

    @pl.when(t == 0)
    def _():
        ssum_ref[...] = jnp.zeros_like(ssum_ref)
        ssq_ref[...] = jnp.zeros_like(ssq_ref)
        ymax_ref[...] = jnp.full_like(ymax_ref, -jnp.inf)
        ymin_ref[...] = jnp.full_like(ymin_ref, jnp.inf)

    ssum_ref[...] += jnp.sum(y, axis=0, keepdims=True)[None]
    ssq_ref[...] += jnp.sum(y * y, axis=0, keepdims=True)[None]
    ymax_ref[...] = jnp.maximum(ymax_ref[...],
                                jnp.max(y, axis=0, keepdims=True)[None])
    ymin_ref[...] = jnp.minimum(ymin_ref[...],
                                jnp.min(y, axis=0, keepdims=True)[None])


def _head1(xc, w5t, MH=512):
    B, N, C = xc.shape
    O = w5t.shape[1]
    T = N // MH
    return pl.pallas_call(
        _h1_kernel,
        grid=(B, T),
        in_specs=[
            pl.BlockSpec((1, MH, C), lambda b, t: (b, t, 0)),
            pl.BlockSpec((C, O), lambda b, t: (0, 0)),
        ],
        out_specs=[
            pl.BlockSpec((1, 1, O), lambda b, t: (b, 0, 0)),
            pl.BlockSpec((1, 1, O), lambda b, t: (b, 0, 0)),
            pl.BlockSpec((1, 1, O), lambda b, t: (b, 0, 0)),
            pl.BlockSpec((1, 1, O), lambda b, t: (b, 0, 0)),
        ],
        out_shape=[jax.ShapeDtypeStruct((B, 1, O), jnp.float32)] * 4,
        compiler_params=pltpu.CompilerParams(
            dimension_semantics=("parallel", "arbitrary"),
            vmem_limit_bytes=100 * 1024 * 1024,
        ),
        name="head1_stats",
    )(xc, w5t)


def _h2_kernel(xc_ref, at_ref, gt_ref, ymax_ref, ymin_ref,
               mean_ref, r_ref, g5_ref, b5_ref,
               hpre_ref, ssum_ref, ssq_ref):
    t = pl.program_id(1)
    mean = mean_ref[...]
    r = r_ref[...]
    sgn = g5_ref[...] * r
    ysel = jnp.where(sgn >= 0, ymax_ref[...], ymin_ref[...])
    gvec = _lrelu((ysel - mean) * r * g5_ref[...] + b5_ref[...])[0]
    c = jnp.dot(gvec, gt_ref[...],
                preferred_element_type=jnp.float32)
    hpre = jnp.dot(xc_ref[0], at_ref[...],
                   preferred_element_type=jnp.float32) + c

    @pl.when(t == 0)
    def _():
        ssum_ref[...] = jnp.zeros_like(ssum_ref)
        ssq_ref[...] = jnp.zeros_like(ssq_ref)

    ssum_ref[...] += jnp.sum(hpre, axis=0, keepdims=True)[None]
    ssq_ref[...] += jnp.sum(hpre * hpre, axis=0, keepdims=True)[None]
    hpre_ref[0] = hpre


def _head2(xc, at, gt, ymax, ymin, st, sq, g5, b5, cnt, MH=512):
    B, N, C = xc.shape
    O = at.shape[1]
    T = N // MH
    mean = st / cnt
    r = jax.lax.rsqrt((sq / cnt - mean * mean) + EPS)
    st, sq = mean, r
    kernel = _h2_kernel
    return pl.pallas_call(
        kernel,
        grid=(B, T),
        in_specs=[
            pl.BlockSpec((1, MH, C), lambda b, t: (b, t, 0)),
            pl.BlockSpec((C, O), lambda b, t: (0, 0)),
            pl.BlockSpec((C, O), lambda b, t: (0, 0)),
            pl.BlockSpec((1, 1, C), lambda b, t: (b, 0, 0)),
            pl.BlockSpec((1, 1, C), lambda b, t: (b, 0, 0)),
            pl.BlockSpec((1, 1, C), lambda b, t: (0, 0, 0)),
            pl.BlockSpec((1, 1, C), lambda b, t: (0, 0, 0)),
            pl.BlockSpec((1, 1, C), lambda b, t: (0, 0, 0)),
            pl.BlockSpec((1, 1, C), lambda b, t: (0, 0, 0)),
        ],
        out_specs=[
            pl.BlockSpec((1, MH, O), lambda b, t: (b, t, 0)),
            pl.BlockSpec((1, 1, O), lambda b, t: (b, 0, 0)),
            pl.BlockSpec((1, 1, O), lambda b, t: (b, 0, 0)),
        ],
        out_shape=[
            jax.ShapeDtypeStruct((B, N, O), jnp.float32),
            jax.ShapeDtypeStruct((B, 1, O), jnp.float32),
            jax.ShapeDtypeStruct((B, 1, O), jnp.float32),
        ],
        compiler_params=pltpu.CompilerParams(
            dimension_semantics=("parallel", "arbitrary"),
            vmem_limit_bytes=100 * 1024 * 1024,
        ),
        name="head2_mlp",
    )(xc, at, gt, ymax, ymin, st, sq, g5, b5)


def _h3_kernel(x_ref, w_ref, mean_ref, r_ref, g_ref, b_ref,
               y_ref, ssum_ref, ssq_ref):
    t = pl.program_id(1)
    mean = mean_ref[...]
    r = r_ref[...]
    h = _lrelu((x_ref[0] - mean[0]) * r[0] * g_ref[0] + b_ref[0])
    y = jnp.dot(h, w_ref[...], preferred_element_type=jnp.float32)

    @pl.when(t == 0)
    def _():
        ssum_ref[...] = jnp.zeros_like(ssum_ref)
        ssq_ref[...] = jnp.zeros_like(ssq_ref)

    ssum_ref[...] += jnp.sum(y, axis=0, keepdims=True)[None]
    ssq_ref[...] += jnp.sum(y * y, axis=0, keepdims=True)[None]
    y_ref[0] = y


def _head3(x, wt, st, sq, g, b, cnt, MH=512):
    B, N, C = x.shape
    O = wt.shape[1]
    T = N // MH
    mean = st / cnt
    r = jax.lax.rsqrt((sq / cnt - mean * mean) + EPS)
    st, sq = mean, r
    kernel = _h3_kernel
    return pl.pallas_call(
        kernel,
        grid=(B, T),
        in_specs=[
            pl.BlockSpec((1, MH, C), lambda b_, t: (b_, t, 0)),
            pl.BlockSpec((C, O), lambda b_, t: (0, 0)),
            pl.BlockSpec((1, 1, C), lambda b_, t: (0, 0, 0)),
            pl.BlockSpec((1, 1, C), lambda b_, t: (0, 0, 0)),
            pl.BlockSpec((1, 1, C), lambda b_, t: (0, 0, 0)),
            pl.BlockSpec((1, 1, C), lambda b_, t: (0, 0, 0)),
        ],
        out_specs=[
            pl.BlockSpec((1, MH, O), lambda b_, t: (b_, t, 0)),
            pl.BlockSpec((1, 1, O), lambda b_, t: (b_, 0, 0)),
            pl.BlockSpec((1, 1, O), lambda b_, t: (b_, 0, 0)),
        ],
        out_shape=[
            jax.ShapeDtypeStruct((B, N, O), jnp.float32),
            jax.ShapeDtypeStruct((B, 1, O), jnp.float32),
            jax.ShapeDtypeStruct((B, 1, O), jnp.float32),
        ],
        compiler_params=pltpu.CompilerParams(
            dimension_semantics=("parallel", "arbitrary"),
            vmem_limit_bytes=100 * 1024 * 1024,
        ),
        name=f"head3_mlp_c{C}",
    )(x, wt, st, sq, g, b)


def _h4_kernel(x_ref, w_ref, mean_ref, r_ref, g_ref, b_ref, bias_ref,
               y_ref):
    mean = mean_ref[...]
    r = r_ref[...]
    h = _lrelu((x_ref[0] - mean[0]) * r[0] * g_ref[0] + b_ref[0])
    y_ref[0] = jnp.dot(h, w_ref[...],
                       preferred_element_type=jnp.float32) + bias_ref[0]


def _head4(x, wt, st, sq, g, b, bias, cnt, MH=512):
    B, N, C = x.shape
    O = wt.shape[1]
    T = N // MH
    mean = st / cnt
    r = jax.lax.rsqrt((sq / cnt - mean * mean) + EPS)
    st, sq = mean, r
    kernel = _h4_kernel
    return pl.pallas_call(
        kernel,
        grid=(B, T),
        in_specs=[
            pl.BlockSpec((1, MH, C), lambda b_, t: (b_, t, 0)),
            pl.BlockSpec((C, O), lambda b_, t: (0, 0)),
            pl.BlockSpec((1, 1, C), lambda b_, t: (0, 0, 0)),
            pl.BlockSpec((1, 1, C), lambda b_, t: (0, 0, 0)),
            pl.BlockSpec((1, 1, C), lambda b_, t: (0, 0, 0)),
            pl.BlockSpec((1, 1, C), lambda b_, t: (0, 0, 0)),
            pl.BlockSpec((1, 1, O), lambda b_, t: (0, 0, 0)),
        ],
        out_specs=pl.BlockSpec((1, MH, O), lambda b_, t: (b_, t, 0)),
        out_shape=jax.ShapeDtypeStruct((B, N, O), jnp.float32),
        compiler_params=pltpu.CompilerParams(
            dimension_semantics=("parallel", "arbitrary"),
            vmem_limit_bytes=100 * 1024 * 1024,
        ),
        name="head4_out",
    )(x, wt, st, sq, g, b, bias)


def kernel(points, W1, g1, b1, W2, g2, b2, W3, g3, b3, W4, g4, b4,
           W5, g5, b5, Ws1, gs1, bs1, Ws2, gs2, bs2, Ws3, bias3):
    B, N, _ = points.shape
    cnt_e = float(B * N * KNN)
    cnt_n = float(B * N)

    xt = points.astype(jnp.float32)

    outs = []
    x_cur = xt
    for (W, g, b) in ((W1, g1, b1), (W2, g2, b2), (W3, g3, b3), (W4, g4, b4)):
        zmax, zmin, ssum, ssq = _edge_block(x_cur, W.T.astype(jnp.float32))
        x_cur = _edge_finish(zmax, zmin, ssum, ssq, g, b, cnt_e)
        outs.append(x_cur)

    xc = jnp.concatenate(outs, axis=2)

    s5sum, s5sq, ymax, ymin = _head1(xc, W5.T.astype(jnp.float32))
    st5 = jnp.sum(s5sum, axis=0).reshape(1, 1, 512)
    sq5 = jnp.sum(s5sq, axis=0).reshape(1, 1, 512)

    at = Ws1[:, :512].T.astype(jnp.float32)
    gt = Ws1[:, 512:].T.astype(jnp.float32)
    hpre, s1sum, s1sq = _head2(
        xc, at, gt, ymax, ymin, st5, sq5,
        g5.reshape(1, 1, 512).astype(jnp.float32),
        b5.reshape(1, 1, 512).astype(jnp.float32), cnt_n)
    st1 = jnp.sum(s1sum, axis=0).reshape(1, 1, 256)
    sq1 = jnp.sum(s1sq, axis=0).reshape(1, 1, 256)

    y2, s2sum, s2sq = _head3(
        hpre, Ws2.T.astype(jnp.float32), st1, sq1,
        gs1.reshape(1, 1, 256).astype(jnp.float32),
        bs1.reshape(1, 1, 256).astype(jnp.float32), cnt_n)
    st2 = jnp.sum(s2sum, axis=0).reshape(1, 1, 128)
    sq2 = jnp.sum(s2sq, axis=0).reshape(1, 1, 128)

    w3pad = jnp.zeros((128, 128), jnp.float32).at[:, :2].set(
        Ws3.T.astype(jnp.float32))
    bias_pad = jnp.zeros((1, 1, 128), jnp.float32).at[0, 0, :2].set(
        bias3.astype(jnp.float32))
    o = _head4(y2, w3pad, st2, sq2,
               gs2.reshape(1, 1, 128).astype(jnp.float32),
               bs2.reshape(1, 1, 128).astype(jnp.float32),
               bias_pad, cnt_n)

    return jnp.transpose(o[:, :, :2], (0, 2, 1))
